# baseline (speedup 1.0000x reference)
import functools

import jax
import jax.numpy as jnp
from jax.experimental import pallas as pl

_N = 10000
_C = 128
_NP = 10240
_KS = (2000, 1000, 500)
_KPS = (2048, 1024, 512)


def _mm_kernel(a_ref, b_ref, s_ref, d_ref, o_ref, *, nk, bm, bn,
               relu, softmax, tanh, zero_diag):
    k = pl.program_id(2)

    @pl.when(k == 0)
    def _init():
        o_ref[...] = jnp.zeros_like(o_ref)

    a = a_ref[...]
    b = b_ref[...]
    if a.dtype != b.dtype:
        a = a.astype(jnp.float32)
        b = b.astype(jnp.float32)
    o_ref[...] += jnp.dot(a, b, preferred_element_type=jnp.float32)

    @pl.when(k == nk - 1)
    def _epilogue():
        acc = s_ref[...] * o_ref[...] + d_ref[...]
        if zero_diag:
            i = pl.program_id(0)
            j = pl.program_id(1)
            rows = jax.lax.broadcasted_iota(jnp.int32, (bm, bn), 0) + i * bm
            cols = jax.lax.broadcasted_iota(jnp.int32, (bm, bn), 1) + j * bn
            acc = jnp.where(rows == cols, 0.0, acc)
        if relu:
            acc = jnp.maximum(acc, 0.0)
        if tanh:
            acc = jnp.tanh(acc)
        if softmax:
            mx = jnp.max(acc, axis=1, keepdims=True)
            e = jnp.exp(acc - mx)
            acc = e / jnp.sum(e, axis=1, keepdims=True)
        o_ref[...] = acc


def _blk(dim, cap):
    return dim if dim <= cap else cap


def _mm(a, b, scale=None, add=None, relu=False, softmax=False, tanh=False,
        zero_diag=False):
    m, kk = a.shape
    _, n = b.shape
    bm, bn, bk = _blk(m, 1024), _blk(n, 1024), _blk(kk, 1024)
    gm, gn, gk = m // bm, n // bn, kk // bk
    if scale is None:
        scale = jnp.ones((1, 1), jnp.float32)
    if add is None:
        add = jnp.zeros((1, 1), jnp.float32)
    sm = bm if scale.shape[0] > 1 else 1
    dm = bm if add.shape[0] > 1 else 1
    dn = bn if add.shape[1] > 1 else 1
    if sm > 1:
        s_idx = lambda i, j, k: (i, 0)
    else:
        s_idx = lambda i, j, k: (0, 0)
    if dm > 1:
        d_idx = lambda i, j, k: (i, j)
    else:
        d_idx = lambda i, j, k: (0, 0)
    kern = functools.partial(_mm_kernel, nk=gk, bm=bm, bn=bn, relu=relu,
                             softmax=softmax, tanh=tanh, zero_diag=zero_diag)
    return pl.pallas_call(
        kern,
        grid=(gm, gn, gk),
        in_specs=[
            pl.BlockSpec((bm, bk), lambda i, j, k: (i, k)),
            pl.BlockSpec((bk, bn), lambda i, j, k: (k, j)),
            pl.BlockSpec((sm, 1), s_idx),
            pl.BlockSpec((dm, dn), d_idx),
        ],
        out_specs=pl.BlockSpec((bm, bn), lambda i, j, k: (i, j)),
        out_shape=jax.ShapeDtypeStruct((m, n), jnp.float32),
    )(a, b, scale, add)


def _gcn(h, A, W, b, dis, fill, relu=False, softmax=False):
    z = _mm(h, W)
    zz = dis[:, None] * z
    add = (dis * dis * fill)[:, None] * z + b[None, :]
    return _mm(A, zz, scale=dis[:, None], add=add, relu=relu, softmax=softmax)


def kernel(x, edge_index, Wd0, bd0, Wd1, bd1, Wd2, bd2, Wd3, bd3,
           pw0, pw1, pw2, Wu0, bu0, Wu1, bu1, Wu2, bu2):
    src = edge_index[0].astype(jnp.int32)
    dst = edge_index[1].astype(jnp.int32)

    A0 = jnp.zeros((_NP, _NP), jnp.bfloat16).at[dst, src].add(1.0)
    deg_e = jnp.zeros((_NP,), jnp.float32).at[dst].add(1.0)
    selfc = jnp.zeros((_NP,), jnp.float32).at[dst].add(
        jnp.where(src == dst, 1.0, 0.0))
    fill0 = jnp.where(selfc == 0.0, 2.0, 0.0)
    deg0 = deg_e + fill0
    dis0 = jnp.where(deg0 > 0.0, jax.lax.rsqrt(deg0), 0.0)

    xp = jnp.zeros((_NP, _C), jnp.float32).at[:_N, :].set(x)
    h0 = _gcn(xp, A0, Wd0, bd0, dis0, fill0, relu=True)

    hs = [h0]
    As = [A0]
    diss = [dis0]
    fills = [fill0]
    perms = []
    pws = (pw0, pw1, pw2)
    Wds = ((Wd1, bd1), (Wd2, bd2), (Wd3, bd3))
    reals = (_N,) + _KS[:-1]

    h, A = h0, A0
    for lvl in range(3):
        k_real, k_pad = _KS[lvl], _KPS[lvl]
        src_real = reals[lvl]
        pw = pws[lvl]
        pwn = (pw / jnp.linalg.norm(pw))[:, None]
        score = _mm(h, pwn, tanh=True)[:src_real, 0]
        sv, perm = jax.lax.top_k(score, k_real)
        perm_p = jnp.concatenate(
            [perm, src_real + jnp.arange(k_pad - k_real, dtype=perm.dtype)])
        rows = A[perm_p, :].at[jnp.arange(k_pad), perm_p].set(1.0)
        cols = A[:, perm_p].at[perm_p, jnp.arange(k_pad)].set(1.0)
        Ap = _mm(rows, cols, zero_diag=True)
        xpool = jnp.zeros((k_pad, _C), jnp.float32).at[:k_real, :].set(
            h[perm] * sv[:, None])
        deg = jnp.sum(Ap, axis=1) + 2.0
        dis = jax.lax.rsqrt(deg)
        fill = jnp.full((k_pad,), 2.0, jnp.float32)
        W, b = Wds[lvl]
        Ap16 = Ap.astype(jnp.bfloat16)
        h = _gcn(xpool, Ap16, W, b, dis, fill, relu=True)
        A = Ap16
        perms.append(perm_p)
        if lvl < 2:
            hs.append(h)
            As.append(Ap16)
            diss.append(dis)
            fills.append(fill)

    Wus = ((Wu0, bu0), (Wu1, bu1), (Wu2, bu2))
    for i in range(3):
        j = 2 - i
        res = hs[j]
        up = jnp.zeros_like(res).at[perms[j], :].set(h)
        hh = res + up
        W, b = Wus[i]
        h = _gcn(hh, As[j], W, b, diss[j], fills[j],
                 relu=(i < 2), softmax=(i == 2))

    return h[:_N, :]


# speedup vs baseline: 1.2085x; 1.2085x over previous
import functools

import jax
import jax.numpy as jnp
from jax.experimental import pallas as pl

_N = 10000
_C = 128
_NP = 10240
_KS = (2000, 1000, 500)
_KPS = (2048, 1024, 512)


def _mm_kernel(a_ref, b_ref, s_ref, d_ref, o_ref, *, nk, bm, bn,
               relu, softmax, tanh, zero_diag):
    k = pl.program_id(2)

    @pl.when(k == 0)
    def _init():
        o_ref[...] = jnp.zeros_like(o_ref)

    a = a_ref[...]
    b = b_ref[...]
    if a.dtype != b.dtype:
        a = a.astype(jnp.float32)
        b = b.astype(jnp.float32)
    o_ref[...] += jnp.dot(a, b, preferred_element_type=jnp.float32)

    @pl.when(k == nk - 1)
    def _epilogue():
        acc = s_ref[...] * o_ref[...] + d_ref[...]
        if zero_diag:
            i = pl.program_id(0)
            j = pl.program_id(1)
            rows = jax.lax.broadcasted_iota(jnp.int32, (bm, bn), 0) + i * bm
            cols = jax.lax.broadcasted_iota(jnp.int32, (bm, bn), 1) + j * bn
            acc = jnp.where(rows == cols, 0.0, acc)
        if relu:
            acc = jnp.maximum(acc, 0.0)
        if tanh:
            acc = jnp.tanh(acc)
        if softmax:
            mx = jnp.max(acc, axis=1, keepdims=True)
            e = jnp.exp(acc - mx)
            acc = e / jnp.sum(e, axis=1, keepdims=True)
        o_ref[...] = acc


def _blk(dim, cap):
    return dim if dim <= cap else cap


def _mm(a, b, scale=None, add=None, relu=False, softmax=False, tanh=False,
        zero_diag=False):
    m, kk = a.shape
    _, n = b.shape
    bm, bn, bk = _blk(m, 1024), _blk(n, 1024), _blk(kk, 1024)
    gm, gn, gk = m // bm, n // bn, kk // bk
    if scale is None:
        scale = jnp.ones((1, 1), jnp.float32)
    if add is None:
        add = jnp.zeros((1, 1), jnp.float32)
    sm = bm if scale.shape[0] > 1 else 1
    dm = bm if add.shape[0] > 1 else 1
    dn = bn if add.shape[1] > 1 else 1
    if sm > 1:
        s_idx = lambda i, j, k: (i, 0)
    else:
        s_idx = lambda i, j, k: (0, 0)
    if dm > 1:
        d_idx = lambda i, j, k: (i, j)
    else:
        d_idx = lambda i, j, k: (0, 0)
    kern = functools.partial(_mm_kernel, nk=gk, bm=bm, bn=bn, relu=relu,
                             softmax=softmax, tanh=tanh, zero_diag=zero_diag)
    return pl.pallas_call(
        kern,
        grid=(gm, gn, gk),
        in_specs=[
            pl.BlockSpec((bm, bk), lambda i, j, k: (i, k)),
            pl.BlockSpec((bk, bn), lambda i, j, k: (k, j)),
            pl.BlockSpec((sm, 1), s_idx),
            pl.BlockSpec((dm, dn), d_idx),
        ],
        out_specs=pl.BlockSpec((bm, bn), lambda i, j, k: (i, j)),
        out_shape=jax.ShapeDtypeStruct((m, n), jnp.float32),
    )(a, b, scale, add)


def _gcn(h, A, W, b, dis, fill, relu=False, softmax=False):
    z = _mm(h, W)
    zz = dis[:, None] * z
    add = (dis * dis * fill)[:, None] * z + b[None, :]
    return _mm(A, zz, scale=dis[:, None], add=add, relu=relu, softmax=softmax)


def kernel(x, edge_index, Wd0, bd0, Wd1, bd1, Wd2, bd2, Wd3, bd3,
           pw0, pw1, pw2, Wu0, bu0, Wu1, bu1, Wu2, bu2):
    src = edge_index[0].astype(jnp.int32)
    dst = edge_index[1].astype(jnp.int32)

    A0 = jnp.zeros((_NP, _NP), jnp.float32).at[dst, src].add(1.0)
    deg_e = jnp.zeros((_NP,), jnp.float32).at[dst].add(1.0)
    selfc = jnp.zeros((_NP,), jnp.float32).at[dst].add(
        jnp.where(src == dst, 1.0, 0.0))
    fill0 = jnp.where(selfc == 0.0, 2.0, 0.0)
    deg0 = deg_e + fill0
    dis0 = jnp.where(deg0 > 0.0, jax.lax.rsqrt(deg0), 0.0)

    xp = jnp.zeros((_NP, _C), jnp.float32).at[:_N, :].set(x)
    h0 = _gcn(xp, A0, Wd0, bd0, dis0, fill0, relu=True)

    hs = [h0]
    As = [A0]
    diss = [dis0]
    fills = [fill0]
    perms = []
    pws = (pw0, pw1, pw2)
    Wds = ((Wd1, bd1), (Wd2, bd2), (Wd3, bd3))
    reals = (_N,) + _KS[:-1]

    h, A = h0, A0
    for lvl in range(3):
        k_real, k_pad = _KS[lvl], _KPS[lvl]
        src_real = reals[lvl]
        pw = pws[lvl]
        pwn = (pw / jnp.linalg.norm(pw))[:, None]
        score = _mm(h, pwn, tanh=True)[:src_real, 0]
        sv, perm = jax.lax.top_k(score, k_real)
        perm_p = jnp.concatenate(
            [perm, src_real + jnp.arange(k_pad - k_real, dtype=perm.dtype)])
        rows = A[perm_p, :].at[jnp.arange(k_pad), perm_p].set(1.0)
        cols = A[:, perm_p].at[perm_p, jnp.arange(k_pad)].set(1.0)
        Ap = _mm(rows.astype(jnp.bfloat16), cols.astype(jnp.bfloat16),
                 zero_diag=True)
        xpool = jnp.zeros((k_pad, _C), jnp.float32).at[:k_real, :].set(
            h[perm] * sv[:, None])
        deg = jnp.sum(Ap, axis=1) + 2.0
        dis = jax.lax.rsqrt(deg)
        fill = jnp.full((k_pad,), 2.0, jnp.float32)
        W, b = Wds[lvl]
        h = _gcn(xpool, Ap, W, b, dis, fill, relu=True)
        A = Ap
        perms.append(perm_p)
        if lvl < 2:
            hs.append(h)
            As.append(Ap)
            diss.append(dis)
            fills.append(fill)

    Wus = ((Wu0, bu0), (Wu1, bu1), (Wu2, bu2))
    for i in range(3):
        j = 2 - i
        res = hs[j]
        up = jnp.zeros_like(res).at[perms[j], :].set(h)
        hh = res + up
        W, b = Wus[i]
        h = _gcn(hh, As[j], W, b, diss[j], fills[j],
                 relu=(i < 2), softmax=(i == 2))

    return h[:_N, :]
